# baseline (speedup 1.0000x reference)
import functools

import jax
import jax.numpy as jnp
from jax import lax
from jax.experimental import pallas as pl
from jax.experimental.pallas import tpu as pltpu
from jax.experimental.pallas import tpu_sc as plsc

B = 1024
P = 16
Q = 16
D = 128
L = 16
NW = 32
EW = B // NW
CH = 8
NCHUNK = EW // CH
NBUF = 3
OUT_ROWS = B * (P + Q) // D

_mesh = plsc.VectorSubcoreMesh(core_axis_name="c", subcore_axis_name="s")

_DNUMS = lax.GatherDimensionNumbers(
    offset_dims=(), collapsed_slice_dims=(0,), start_index_map=(0,))


def _perm(x, pm):
    return lax.gather(x, pm, _DNUMS, slice_sizes=(1,),
                      mode=lax.GatherScatterMode.PROMISE_IN_BOUNDS)


@functools.partial(
    pl.kernel,
    out_type=jax.ShapeDtypeStruct((OUT_ROWS, D), jnp.float32),
    mesh=_mesh,
    scratch_types=[
        pltpu.VMEM((EW,), jnp.int32),
        pltpu.VMEM((EW, P), jnp.int32),
        pltpu.VMEM((EW, Q), jnp.int32),
        pltpu.VMEM((NCHUNK, CH * P), jnp.int32),
        pltpu.VMEM((NCHUNK, CH * Q), jnp.int32),
        pltpu.VMEM((EW, D), jnp.float32),
        pltpu.VMEM((CH * P, D), jnp.float32),
        pltpu.VMEM((CH * P, D), jnp.float32),
        pltpu.VMEM((CH * P, D), jnp.float32),
        pltpu.VMEM((CH * Q, D), jnp.float32),
        pltpu.VMEM((CH * Q, D), jnp.float32),
        pltpu.VMEM((CH * Q, D), jnp.float32),
        pltpu.VMEM((EW * (P + Q) // D, D), jnp.float32),
        pltpu.SemaphoreType.DMA,
        pltpu.SemaphoreType.DMA,
        pltpu.SemaphoreType.DMA,
        pltpu.SemaphoreType.DMA,
        pltpu.SemaphoreType.DMA,
        pltpu.SemaphoreType.DMA,
        pltpu.SemaphoreType.DMA,
    ],
)
def _sc_logits(emb_hbm, nidx_hbm, pidx_hbm, qidx_hbm, out_hbm,
               nidx_v, pstage_v, qstage_v, pidx_v, qidx_v, a_rows,
               p0, p1, p2, q0, q1, q2,
               logit_v, sem_a, sp0, sp1, sp2, sq0, sq1, sq2):
    wid = lax.axis_index("s") * 2 + lax.axis_index("c")
    base = wid * EW
    pltpu.sync_copy(nidx_hbm.at[pl.ds(base, EW)], nidx_v)
    pltpu.sync_copy(pidx_hbm.at[pl.ds(base, EW), :], pstage_v)
    pltpu.sync_copy(qidx_hbm.at[pl.ds(base, EW), :], qstage_v)
    ha = pltpu.async_copy(emb_hbm.at[nidx_v], a_rows, sem_a)
    for c in range(NCHUNK):
        for j in range(CH):
            pidx_v[c, pl.ds(j * P, P)] = pstage_v[c * CH + j, :]
            qidx_v[c, pl.ds(j * Q, Q)] = qstage_v[c * CH + j, :]

    pbuf, qbuf = [p0, p1, p2], [q0, q1, q2]
    psem, qsem = [sp0, sp1, sp2], [sq0, sq1, sq2]
    hp, hq = [None] * NCHUNK, [None] * NCHUNK

    def issue(c):
        hp[c] = pltpu.async_copy(
            emb_hbm.at[pidx_v.at[c]], pbuf[c % NBUF], psem[c % NBUF])
        hq[c] = pltpu.async_copy(
            emb_hbm.at[qidx_v.at[c]], qbuf[c % NBUF], qsem[c % NBUF])

    for c in range(min(NBUF, NCHUNK)):
        issue(c)
    ha.wait()

    lanes = lax.iota(jnp.int32, L)
    shifts = (1, 2, 4, 8)
    masks = [(lanes & sh) == 0 for sh in shifts]
    perms = [(lanes ^ sh)[:, None] for sh in shifts]

    for c in range(NCHUNK):
        hp[c].wait()
        hq[c].wait()
        pb, qb = pbuf[c % NBUF], qbuf[c % NBUF]

        def elem_body(e, carry, c=c, pb=pb, qb=qb):
            ee = c * CH + e
            a = [a_rows[ee, pl.ds(L * k, L)] for k in range(D // L)]

            def side(buf):
                u = []
                for n in range(P):
                    r = e * P + n
                    s = buf[r, pl.ds(0, L)] * a[0]
                    for k in range(1, D // L):
                        s = s + buf[r, pl.ds(L * k, L)] * a[k]
                    u.append(s)
                for m, pm in zip(masks, perms):
                    u = [jnp.where(m, u[2 * i], u[2 * i + 1])
                         + _perm(jnp.where(m, u[2 * i + 1], u[2 * i]), pm)
                         for i in range(len(u) // 2)]
                return u[0]

            vp = side(pb)
            vq = side(qb)
            row = ee // 4
            colbase = (ee % 4) * (P + Q)
            logit_v[row, pl.ds(colbase, L)] = vp
            logit_v[row, pl.ds(colbase + P, L)] = vq
            return carry

        lax.fori_loop(0, CH, elem_body, 0)
        if c + NBUF < NCHUNK:
            issue(c + NBUF)

    pltpu.sync_copy(logit_v, out_hbm.at[pl.ds(wid * (EW * (P + Q) // D),
                                              EW * (P + Q) // D)])


def _tc_body(logit_ref, out_ref):
    x = logit_ref[...]
    col = lax.broadcasted_iota(jnp.int32, x.shape, 1)
    z = jnp.where((col & P) == 0, -x, x)
    sp = jnp.maximum(z, 0.0) + jnp.log1p(jnp.exp(-jnp.abs(z)))
    out_ref[0, 0] = jnp.sum(sp) * (1.0 / P)


_tc_loss = pl.pallas_call(
    _tc_body,
    out_shape=jax.ShapeDtypeStruct((1, 1), jnp.float32),
    out_specs=pl.BlockSpec(memory_space=pltpu.SMEM),
)


def kernel(embeddings, node_idx, pos_idx, neg_idx):
    logits = _sc_logits(embeddings,
                        node_idx.astype(jnp.int32),
                        pos_idx.astype(jnp.int32),
                        neg_idx.astype(jnp.int32))
    return _tc_loss(logits).reshape(())


# speedup vs baseline: 1.0807x; 1.0807x over previous
import functools

import jax
import jax.numpy as jnp
from jax import lax
from jax.experimental import pallas as pl
from jax.experimental.pallas import tpu as pltpu
from jax.experimental.pallas import tpu_sc as plsc

B = 1024
P = 16
Q = 16
D = 128
L = 16
NW = 32
EW = B // NW
CH = 8
NCHUNK = EW // CH
ROWS = CH * P
HALF = ROWS // 2
IDXW = EW * (1 + P + Q)
OUT_ROWS = B * (P + Q) // D

_mesh = plsc.VectorSubcoreMesh(core_axis_name="c", subcore_axis_name="s")

_DNUMS = lax.GatherDimensionNumbers(
    offset_dims=(), collapsed_slice_dims=(0,), start_index_map=(0,))


def _perm(x, pm):
    return lax.gather(x, pm, _DNUMS, slice_sizes=(1,),
                      mode=lax.GatherScatterMode.PROMISE_IN_BOUNDS)


@functools.partial(
    pl.kernel,
    out_type=jax.ShapeDtypeStruct((OUT_ROWS, D), jnp.float32),
    mesh=_mesh,
    scratch_types=[
        pltpu.VMEM((IDXW,), jnp.int32),
        pltpu.VMEM((EW, D), jnp.float32),
        pltpu.VMEM((ROWS, D), jnp.float32),
        pltpu.VMEM((ROWS, D), jnp.float32),
        pltpu.VMEM((ROWS, D), jnp.float32),
        pltpu.VMEM((ROWS, D), jnp.float32),
        pltpu.VMEM((EW * (P + Q) // D, D), jnp.float32),
        pltpu.SemaphoreType.DMA,
        pltpu.SemaphoreType.DMA,
        pltpu.SemaphoreType.DMA,
        pltpu.SemaphoreType.DMA,
        pltpu.SemaphoreType.DMA,
        pltpu.SemaphoreType.DMA,
        pltpu.SemaphoreType.DMA,
        pltpu.SemaphoreType.DMA,
        pltpu.SemaphoreType.DMA,
    ],
)
def _sc_logits(emb_hbm, idx_hbm, out_hbm,
               idx_v, a_rows, p0, p1, q0, q1, logit_v,
               sem_a, sp0a, sp0b, sp1a, sp1b, sq0a, sq0b, sq1a, sq1b):
    wid = lax.axis_index("s") * 2 + lax.axis_index("c")
    pltpu.sync_copy(idx_hbm.at[pl.ds(wid * IDXW, IDXW)], idx_v)
    ha = pltpu.async_copy(emb_hbm.at[idx_v.at[pl.ds(0, EW)]], a_rows, sem_a)

    pbuf, qbuf = [p0, p1], [q0, q1]
    psem = [(sp0a, sp0b), (sp1a, sp1b)]
    qsem = [(sq0a, sq0b), (sq1a, sq1b)]
    hs = [None] * NCHUNK

    def issue(c):
        par = c % 2
        pofs = EW + c * ROWS
        qofs = EW + EW * P + c * ROWS
        hs[c] = [
            pltpu.async_copy(emb_hbm.at[idx_v.at[pl.ds(pofs, HALF)]],
                             pbuf[par].at[pl.ds(0, HALF)], psem[par][0]),
            pltpu.async_copy(emb_hbm.at[idx_v.at[pl.ds(pofs + HALF, HALF)]],
                             pbuf[par].at[pl.ds(HALF, HALF)], psem[par][1]),
            pltpu.async_copy(emb_hbm.at[idx_v.at[pl.ds(qofs, HALF)]],
                             qbuf[par].at[pl.ds(0, HALF)], qsem[par][0]),
            pltpu.async_copy(emb_hbm.at[idx_v.at[pl.ds(qofs + HALF, HALF)]],
                             qbuf[par].at[pl.ds(HALF, HALF)], qsem[par][1]),
        ]

    issue(0)
    issue(1)
    ha.wait()

    lanes = lax.iota(jnp.int32, L)
    shifts = (1, 2, 4, 8)
    masks = [(lanes & sh) == 0 for sh in shifts]
    perms = [(lanes ^ sh)[:, None] for sh in shifts]

    for c in range(NCHUNK):
        for h in hs[c]:
            h.wait()
        pb, qb = pbuf[c % 2], qbuf[c % 2]

        def elem_body(e, carry, c=c, pb=pb, qb=qb):
            ee = c * CH + e
            a = [a_rows[ee, pl.ds(L * k, L)] for k in range(D // L)]

            def side(buf):
                u = []
                for n in range(P):
                    r = e * P + n
                    s = buf[r, pl.ds(0, L)] * a[0]
                    for k in range(1, D // L):
                        s = s + buf[r, pl.ds(L * k, L)] * a[k]
                    u.append(s)
                for m, pm in zip(masks, perms):
                    u = [jnp.where(m, u[2 * i], u[2 * i + 1])
                         + _perm(jnp.where(m, u[2 * i + 1], u[2 * i]), pm)
                         for i in range(len(u) // 2)]
                return u[0]

            vp = side(pb)
            vq = side(qb)
            row = ee // 4
            colbase = (ee % 4) * (P + Q)
            logit_v[row, pl.ds(colbase, L)] = vp
            logit_v[row, pl.ds(colbase + P, L)] = vq
            return carry

        lax.fori_loop(0, CH, elem_body, 0)
        if c + 2 < NCHUNK:
            issue(c + 2)

    pltpu.sync_copy(logit_v, out_hbm.at[pl.ds(wid * (EW * (P + Q) // D),
                                              EW * (P + Q) // D)])


def _tc_body(logit_ref, out_ref):
    x = logit_ref[...]
    col = lax.broadcasted_iota(jnp.int32, x.shape, 1)
    z = jnp.where((col & P) == 0, -x, x)
    sp = jnp.maximum(z, 0.0) + jnp.log1p(jnp.exp(-jnp.abs(z)))
    out_ref[0, 0] = jnp.sum(sp) * (1.0 / P)


_tc_loss = pl.pallas_call(
    _tc_body,
    out_shape=jax.ShapeDtypeStruct((1, 1), jnp.float32),
    out_specs=pl.BlockSpec(memory_space=pltpu.SMEM),
)


def kernel(embeddings, node_idx, pos_idx, neg_idx):
    cat = jnp.concatenate([
        node_idx.astype(jnp.int32).reshape(NW, EW),
        pos_idx.astype(jnp.int32).reshape(NW, EW * P),
        neg_idx.astype(jnp.int32).reshape(NW, EW * Q),
    ], axis=1).reshape(-1)
    logits = _sc_logits(embeddings, cat)
    return _tc_loss(logits).reshape(())
